# baseline (speedup 1.0000x reference)
import functools

import jax
import jax.numpy as jnp
from jax import lax
from jax.experimental import pallas as pl
from jax.experimental.pallas import tpu as pltpu
from jax.experimental.pallas import tpu_sc as plsc

_VOCAB = 1000000
_D = 64
_B = 4096
_L = 60
_Q = 50

_NC = 2
_NS = 16
_NW = _NC * _NS
_TOTAL = _B * _Q
_PPT = _TOTAL // _NS
_CPC = _D // _NC
_ROWS_W = _B // _NW
_TPW = _ROWS_W * _L
_MASK_W = _ROWS_W * _Q

_mesh = plsc.VectorSubcoreMesh(
    core_axis_name="c", subcore_axis_name="s", num_cores=_NC, num_subcores=_NS
)


def _emb_body(tab_hbm, idx_hbm, types_hbm, out_hbm, mask_hbm,
              idx_v, types_v, mask_v, buf, spm, sem_g, sem_s):
  sid = lax.axis_index("s")
  cid = lax.axis_index("c")
  wid = sid * _NC + cid
  base = cid * _CPC

  pltpu.sync_copy(idx_hbm.at[sid], idx_v)

  def _stage(row):
    pltpu.async_copy(tab_hbm.at[row], spm, sem_s)

  def _stage_wait(row):
    pltpu.make_async_copy(tab_hbm.at[row], spm, sem_s).wait()

  @pl.when(sid == 0)
  def _():
    _stage(base)

  pltpu.sync_copy(
      types_hbm.at[pl.ds((_L - _Q) * _B + wid * _MASK_W, _MASK_W)], types_v)
  v_two = jnp.full((16,), 2, jnp.int32)
  v_one = jnp.full((16,), 1.0, jnp.float32)
  v_zero = jnp.full((16,), 0.0, jnp.float32)

  def mask_step(i, carry):
    for j in range(4):
      m0 = (i * 4 + j) * 16
      t = types_v[pl.ds(m0, 16)]
      mask_v[pl.ds(m0, 16)] = jnp.where(t == v_two, v_one, v_zero)
    return carry

  lax.fori_loop(0, _MASK_W // 64, mask_step, 0)
  pltpu.sync_copy(mask_v, mask_hbm.at[pl.ds(wid * _MASK_W, _MASK_W)])

  def col_step(c, carry):
    row = base + c

    @pl.when(sid == 0)
    def _():
      _stage_wait(row)

    plsc.subcore_barrier()
    for g in range(4):
      pltpu.async_copy(
          spm.at[idx_v.at[pl.ds(g * 3200, 3200)]],
          buf.at[pl.ds(g * 3200, 3200)], sem_g)
    for g in range(4):
      pltpu.make_async_copy(
          spm.at[idx_v.at[pl.ds(g * 3200, 3200)]],
          buf.at[pl.ds(g * 3200, 3200)], sem_g).wait()
    plsc.subcore_barrier()

    @pl.when((sid == 0) & (c < _CPC - 1))
    def _():
      _stage(row + 1)

    pltpu.sync_copy(buf, out_hbm.at[row].at[pl.ds(sid * _PPT, _PPT)])
    return carry

  lax.fori_loop(0, _CPC, col_step, 0)


_emb_call = functools.partial(
    pl.kernel,
    out_type=[
        jax.ShapeDtypeStruct((_D, _TOTAL), jnp.float32),
        jax.ShapeDtypeStruct((_TOTAL,), jnp.float32),
    ],
    mesh=_mesh,
    scratch_types=[
        pltpu.VMEM((_PPT,), jnp.int32),
        pltpu.VMEM((_MASK_W,), jnp.int32),
        pltpu.VMEM((_MASK_W,), jnp.float32),
        pltpu.VMEM((_PPT,), jnp.float32),
        pltpu.VMEM_SHARED((_VOCAB,), jnp.float32),
        pltpu.SemaphoreType.DMA,
        pltpu.SemaphoreType.DMA,
    ],
)(_emb_body)


def kernel(positions, types, object_positions, object_colors, object_shapes,
           object_materials, object_sizes, question, question_embeddings):
  tab_t = question_embeddings.T
  idx2 = question.T.reshape(_NS, _PPT)
  types_flat = types.T.reshape(-1)
  out, mask = _emb_call(tab_t, idx2, types_flat)
  out3 = out.reshape(_D, _Q, _B)
  return (
      out3.transpose(2, 1, 0),
      mask.reshape(_Q, _B).T.reshape(_B, 1, 1, _Q),
  )


# speedup vs baseline: 1.0007x; 1.0007x over previous
import functools

import jax
import jax.numpy as jnp
from jax import lax
from jax.experimental import pallas as pl
from jax.experimental.pallas import tpu as pltpu
from jax.experimental.pallas import tpu_sc as plsc

_VOCAB = 1000000
_D = 64
_B = 4096
_L = 60
_Q = 50

_NC = 2
_NS = 16
_NW = _NC * _NS
_TOTAL = _B * _Q
_PPT = _TOTAL // _NS
_CPC = _D // _NC
_MASK_W = _TOTAL // _NW

_mesh = plsc.VectorSubcoreMesh(
    core_axis_name="c", subcore_axis_name="s", num_cores=_NC, num_subcores=_NS
)


def _emb_body(tab_hbm, idx_hbm, types_hbm, out_hbm, mask_hbm,
              idx_v, types_v, mask_v, buf, spm, sem_g, sem_s):
  sid = lax.axis_index("s")
  cid = lax.axis_index("c")
  wid = sid * _NC + cid
  base = cid * _CPC

  pltpu.sync_copy(idx_hbm.at[sid], idx_v)

  @pl.when(sid == 0)
  def _():
    pltpu.async_copy(tab_hbm.at[base], spm, sem_s)

  pltpu.sync_copy(
      types_hbm.at[pl.ds((_L - _Q) * _B + wid * _MASK_W, _MASK_W)], types_v)
  v_two = jnp.full((16,), 2, jnp.int32)
  v_one = jnp.full((16,), 1.0, jnp.float32)
  v_zero = jnp.full((16,), 0.0, jnp.float32)

  def mask_step(i, carry):
    for j in range(4):
      m0 = (i * 4 + j) * 16
      t = types_v[pl.ds(m0, 16)]
      mask_v[pl.ds(m0, 16)] = jnp.where(t == v_two, v_one, v_zero)
    return carry

  lax.fori_loop(0, _MASK_W // 64, mask_step, 0)
  pltpu.sync_copy(mask_v, mask_hbm.at[pl.ds(wid * _MASK_W, _MASK_W)])

  def col_step(c, carry):
    row = base + c

    @pl.when(sid == 0)
    def _():
      pltpu.make_async_copy(tab_hbm.at[row], spm, sem_s).wait()

    plsc.subcore_barrier()
    pltpu.async_copy(spm.at[idx_v], buf, sem_g).wait()
    plsc.subcore_barrier()

    @pl.when((sid == 0) & (c < _CPC - 1))
    def _():
      pltpu.async_copy(tab_hbm.at[row + 1], spm, sem_s)

    pltpu.sync_copy(buf, out_hbm.at[row].at[pl.ds(sid * _PPT, _PPT)])
    return carry

  lax.fori_loop(0, _CPC, col_step, 0)


_emb_call = functools.partial(
    pl.kernel,
    out_type=[
        jax.ShapeDtypeStruct((_D, _TOTAL), jnp.float32),
        jax.ShapeDtypeStruct((_TOTAL,), jnp.float32),
    ],
    mesh=_mesh,
    scratch_types=[
        pltpu.VMEM((_PPT,), jnp.int32),
        pltpu.VMEM((_MASK_W,), jnp.int32),
        pltpu.VMEM((_MASK_W,), jnp.float32),
        pltpu.VMEM((_PPT,), jnp.float32),
        pltpu.VMEM_SHARED((_VOCAB,), jnp.float32),
        pltpu.SemaphoreType.DMA,
        pltpu.SemaphoreType.DMA,
    ],
)(_emb_body)


def kernel(positions, types, object_positions, object_colors, object_shapes,
           object_materials, object_sizes, question, question_embeddings):
  tab_t = question_embeddings.T
  idx2 = question.T.reshape(_NS, _PPT)
  types_flat = types.T.reshape(-1)
  out, mask = _emb_call(tab_t, idx2, types_flat)
  out3 = out.reshape(_D, _Q, _B)
  return (
      out3.transpose(2, 1, 0),
      mask.reshape(_Q, _B).T.reshape(_B, 1, 1, _Q),
  )


# speedup vs baseline: 1.1587x; 1.1579x over previous
import functools

import jax
import jax.numpy as jnp
from jax import lax
from jax.experimental import pallas as pl
from jax.experimental.pallas import tpu as pltpu
from jax.experimental.pallas import tpu_sc as plsc

_VOCAB = 1000000
_D = 64
_B = 4096
_L = 60
_Q = 50

_NC = 2
_NS = 16
_NW = _NC * _NS
_TOTAL = _B * _Q
_PPT = _TOTAL // _NS
_CPC = _D // _NC
_MASK_W = _TOTAL // _NW

_mesh = plsc.VectorSubcoreMesh(
    core_axis_name="c", subcore_axis_name="s", num_cores=_NC, num_subcores=_NS
)


def _emb_body(tab_hbm, idx_hbm, types_hbm, out_hbm, mask_hbm,
              idx_v, types_v, mask_v, buf, spm, sem_g, sem_s):
  sid = lax.axis_index("s")
  cid = lax.axis_index("c")
  wid = sid * _NC + cid
  base = cid * _CPC

  q0_a = sid * 3
  q0_b = 42 + (sid - 14) * 4

  @pl.when(sid < 14)
  def _():
    pltpu.sync_copy(idx_hbm.at[pl.ds(q0_a * _B, 3 * _B)],
                    idx_v.at[pl.ds(0, 3 * _B)])

  @pl.when(sid >= 14)
  def _():
    pltpu.sync_copy(idx_hbm.at[pl.ds(q0_b * _B, 4 * _B)],
                    idx_v.at[pl.ds(0, 4 * _B)])

  @pl.when(sid == 0)
  def _():
    pltpu.async_copy(tab_hbm.at[base], spm, sem_s)

  pltpu.sync_copy(
      types_hbm.at[pl.ds((_L - _Q) * _B + wid * _MASK_W, _MASK_W)], types_v)
  v_two = jnp.full((16,), 2, jnp.int32)
  v_one = jnp.full((16,), 1.0, jnp.float32)
  v_zero = jnp.full((16,), 0.0, jnp.float32)

  def mask_step(i, carry):
    for j in range(4):
      m0 = (i * 4 + j) * 16
      t = types_v[pl.ds(m0, 16)]
      mask_v[pl.ds(m0, 16)] = jnp.where(t == v_two, v_one, v_zero)
    return carry

  lax.fori_loop(0, _MASK_W // 64, mask_step, 0)
  pltpu.sync_copy(mask_v, mask_hbm.at[pl.ds(wid * _MASK_W, _MASK_W)])

  def col_step(c, carry):
    row = base + c

    @pl.when(sid == 0)
    def _():
      pltpu.make_async_copy(tab_hbm.at[row], spm, sem_s).wait()

    plsc.subcore_barrier()

    @pl.when(sid < 14)
    def _():
      pltpu.async_copy(spm.at[idx_v.at[pl.ds(0, 3 * _B)]],
                       buf.at[pl.ds(0, 3 * _B)], sem_g).wait()
      for k in range(3):
        pltpu.sync_copy(buf.at[pl.ds(k * _B, _B)],
                        out_hbm.at[q0_a + k].at[row])

    @pl.when(sid >= 14)
    def _():
      pltpu.async_copy(spm.at[idx_v.at[pl.ds(0, 4 * _B)]],
                       buf.at[pl.ds(0, 4 * _B)], sem_g).wait()
      for k in range(4):
        pltpu.sync_copy(buf.at[pl.ds(k * _B, _B)],
                        out_hbm.at[q0_b + k].at[row])

    plsc.subcore_barrier()

    @pl.when((sid == 0) & (c < _CPC - 1))
    def _():
      pltpu.async_copy(tab_hbm.at[row + 1], spm, sem_s)
    return carry

  lax.fori_loop(0, _CPC, col_step, 0)


_emb_call = functools.partial(
    pl.kernel,
    out_type=[
        jax.ShapeDtypeStruct((_Q, _D, _B), jnp.float32),
        jax.ShapeDtypeStruct((_TOTAL,), jnp.float32),
    ],
    mesh=_mesh,
    scratch_types=[
        pltpu.VMEM((4 * _B,), jnp.int32),
        pltpu.VMEM((_MASK_W,), jnp.int32),
        pltpu.VMEM((_MASK_W,), jnp.float32),
        pltpu.VMEM((4 * _B,), jnp.float32),
        pltpu.VMEM_SHARED((_VOCAB,), jnp.float32),
        pltpu.SemaphoreType.DMA,
        pltpu.SemaphoreType.DMA,
    ],
)(_emb_body)


def kernel(positions, types, object_positions, object_colors, object_shapes,
           object_materials, object_sizes, question, question_embeddings):
  tab_t = question_embeddings.T
  idx_flat = question.T.reshape(-1)
  types_flat = types.T.reshape(-1)
  out4, mask = _emb_call(tab_t, idx_flat, types_flat)
  return (
      out4.transpose(2, 0, 1),
      mask.reshape(_Q, _B).T.reshape(_B, 1, 1, _Q),
  )
